# baseline (speedup 1.0000x reference)
import functools

import jax
import jax.numpy as jnp
from jax import lax
from jax.experimental import pallas as pl
from jax.experimental.pallas import tpu as pltpu
from jax.experimental.pallas import tpu_sc as plsc

D = 128
BATCH = 4096
SEQ = 200
B = BATCH * SEQ
L = 16
NC = 2
NS = 16
NW = NC * NS
B_PER_W = B // NW
GI = 128
CH = 256
N_CHUNK = B_PER_W // CH


def _rsqrt16(x):
    i = plsc.bitcast(x, jnp.int32)
    i = jnp.int32(0x5F3759DF) - (i >> 1)
    y = plsc.bitcast(i, jnp.float32)
    half_x = x * 0.5
    for _ in range(2):
        y = y * (1.5 - half_x * y * y)
    return y


@functools.partial(
    pl.kernel,
    out_type=jax.ShapeDtypeStruct((B, D), jnp.float32),
    mesh=plsc.VectorSubcoreMesh(core_axis_name="c", subcore_axis_name="s"),
    scratch_types=[
        pltpu.VMEM((B_PER_W,), jnp.int32),
        [pltpu.VMEM((CH, D), jnp.float32)] * 3,
        [pltpu.SemaphoreType.DMA] * 3,
        [pltpu.SemaphoreType.DMA] * 3,
    ],
    compiler_params=pltpu.CompilerParams(needs_layout_passes=False),
)
def _gather_norm(idx_hbm, table_hbm, out_hbm, idx_all, bufs, gsems, wsems):
    wid = lax.axis_index("s") * NC + lax.axis_index("c")
    base = wid * B_PER_W

    pltpu.sync_copy(idx_hbm.at[pl.ds(base, B_PER_W)], idx_all)

    def gather_start(g, b):
        for h in range(CH // GI):
            pltpu.async_copy(
                table_hbm.at[idx_all.at[pl.ds(g * CH + h * GI, GI)]],
                bufs[b].at[pl.ds(h * GI, GI)], gsems[b])

    def gather_wait(b):
        for h in range(CH // GI):
            pltpu.make_async_copy(
                table_hbm.at[idx_all.at[pl.ds(0, GI)]],
                bufs[b].at[pl.ds(h * GI, GI)], gsems[b]).wait()

    def write_start(g, b):
        pltpu.async_copy(
            bufs[b], out_hbm.at[pl.ds(base + g * CH, CH)], wsems[b])

    def write_wait(b):
        pltpu.make_async_copy(
            bufs[b], out_hbm.at[pl.ds(base, CH)], wsems[b]).wait()

    def normalize(buf):
        def row_body(r):
            vs = []
            acc = None
            for j in range(D // L):
                v = buf[r, pl.ds(j * L, L)]
                vs.append(v)
                acc = v * v if acc is None else acc + v * v
            total = jnp.maximum(jnp.sum(acc), jnp.float32(1e-24))
            inv = _rsqrt16(jnp.full((L,), total, jnp.float32))
            for j in range(D // L):
                buf[r, pl.ds(j * L, L)] = vs[j] * inv

        plsc.parallel_loop(0, CH, unroll=2)(row_body)

    gather_start(0, 0)
    gather_start(1, 1)

    def iter_body(g, b):
        bn = (b + 2) % 3

        @pl.when(g >= 1)
        def _():
            write_wait(bn)

        @pl.when(g < N_CHUNK - 2)
        def _():
            gather_start(g + 2, bn)

        gather_wait(b)
        normalize(bufs[b])
        write_start(g, b)

    def outer_body(o, carry):
        for b in range(3):
            iter_body(3 * o + b, b)
        return carry

    lax.fori_loop(0, N_CHUNK // 3, outer_body, 0)
    iter_body(N_CHUNK - 1, (N_CHUNK - 1) % 3)
    write_wait((N_CHUNK - 1) % 3)


def kernel(indices, weight):
    idx = indices.reshape(-1).astype(jnp.int32)
    out = _gather_norm(idx, weight)
    return out.reshape(BATCH, SEQ, D)


# speedup vs baseline: 1.1162x; 1.1162x over previous
import functools

import jax
import jax.numpy as jnp
from jax import lax
from jax.experimental import pallas as pl
from jax.experimental.pallas import tpu as pltpu
from jax.experimental.pallas import tpu_sc as plsc

D = 128
BATCH = 4096
SEQ = 200
B = BATCH * SEQ
L = 16
NC = 2
NS = 16
NW = NC * NS
B_PER_W = B // NW
CH = 128
N_CHUNK = B_PER_W // CH


def _rsqrt16(x):
    i = plsc.bitcast(x, jnp.int32)
    i = jnp.int32(0x5F3759DF) - (i >> 1)
    y = plsc.bitcast(i, jnp.float32)
    half_x = x * 0.5
    for _ in range(2):
        y = y * (1.5 - half_x * y * y)
    return y


@functools.partial(
    pl.kernel,
    out_type=jax.ShapeDtypeStruct((B, D), jnp.float32),
    mesh=plsc.VectorSubcoreMesh(core_axis_name="c", subcore_axis_name="s"),
    scratch_types=[
        pltpu.VMEM((B_PER_W,), jnp.int32),
        [pltpu.VMEM((CH, D), jnp.float32)] * 5,
        [pltpu.SemaphoreType.DMA] * 5,
        [pltpu.SemaphoreType.DMA] * 5,
    ],
    compiler_params=pltpu.CompilerParams(needs_layout_passes=False),
)
def _gather_norm(idx_hbm, table_hbm, out_hbm, idx_all, bufs, gsems, wsems):
    wid = lax.axis_index("s") * NC + lax.axis_index("c")
    base = wid * B_PER_W

    pltpu.sync_copy(idx_hbm.at[pl.ds(base, B_PER_W)], idx_all)

    def gather_start(g, b):
        pltpu.async_copy(
            table_hbm.at[idx_all.at[pl.ds(g * CH, CH)]], bufs[b], gsems[b])

    def gather_wait(b):
        pltpu.make_async_copy(
            table_hbm.at[idx_all.at[pl.ds(0, CH)]], bufs[b], gsems[b]).wait()

    def write_start(g, b):
        pltpu.async_copy(
            bufs[b], out_hbm.at[pl.ds(base + g * CH, CH)], wsems[b])

    def write_wait(b):
        pltpu.make_async_copy(
            bufs[b], out_hbm.at[pl.ds(base, CH)], wsems[b]).wait()

    def normalize(buf):
        def row_body(r):
            vs = []
            acc = None
            for j in range(D // L):
                v = buf[r, pl.ds(j * L, L)]
                vs.append(v)
                acc = v * v if acc is None else acc + v * v
            total = jnp.maximum(jnp.sum(acc), jnp.float32(1e-24))
            inv = _rsqrt16(jnp.full((L,), total, jnp.float32))
            for j in range(D // L):
                buf[r, pl.ds(j * L, L)] = vs[j] * inv

        plsc.parallel_loop(0, CH, unroll=2)(row_body)

    gather_start(0, 0)
    gather_start(1, 1)

    def outer_body(o, carry):
        for b in range(5):
            g = 5 * o + b
            bn = (b + 2) % 5

            @pl.when(g >= 3)
            def _():
                write_wait(bn)

            @pl.when(g < N_CHUNK - 2)
            def _():
                gather_start(g + 2, bn)

            gather_wait(b)
            normalize(bufs[b])
            write_start(g, b)
        return carry

    lax.fori_loop(0, N_CHUNK // 5, outer_body, 0)
    write_wait(2)
    write_wait(3)
    write_wait(4)


def kernel(indices, weight):
    idx = indices.reshape(-1).astype(jnp.int32)
    out = _gather_norm(idx, weight)
    return out.reshape(BATCH, SEQ, D)
